# Initial kernel scaffold; baseline (speedup 1.0000x reference)
import functools

import jax
import jax.numpy as jnp
from jax import lax
from jax.experimental import pallas as pl
from jax.experimental.pallas import tpu as pltpu
from jax.experimental.pallas import tpu_sc as plsc

B, N, F, H = 256, 128, 9, 128
NUM_IN_DEG = 512

NW = 32
BATCHES_PER_W = B // NW
NODES_PER_W = BATCHES_PER_W * N
CHUNK = 8
CHUNKS_PER_BATCH = N // CHUNK
AROWS = CHUNK * F
DROWS = CHUNK * 2
NCOL = H // 16


def _body(atom_hbm, degc_hbm, gt_hbm, aidx_hbm, didx_hbm, out_hbm,
          aidx_v, didx_v, arows_v, drows_v, out_v, sem_a, sem_d):
    nc = 2
    wid = lax.axis_index("s") * nc + lax.axis_index("c")

    pltpu.sync_copy(aidx_hbm.at[pl.ds(wid * NODES_PER_W * F, NODES_PER_W * F)],
                    aidx_v)
    pltpu.sync_copy(didx_hbm.at[pl.ds(wid * NODES_PER_W * 2, NODES_PER_W * 2)],
                    didx_v)
    pltpu.sync_copy(gt_hbm, out_v.at[pl.ds(0, 1), :])

    def batch_body(lb, carry):
        def chunk_body(c, carry2):
            off = lb * N + c * CHUNK
            pltpu.async_copy(atom_hbm.at[aidx_v.at[pl.ds(off * F, AROWS)]],
                             arows_v, sem_a)
            pltpu.async_copy(degc_hbm.at[didx_v.at[pl.ds(off * 2, DROWS)]],
                             drows_v, sem_d)
            pltpu.make_async_copy(atom_hbm.at[aidx_v.at[pl.ds(off * F, AROWS)]],
                                  arows_v, sem_a).wait()
            pltpu.make_async_copy(degc_hbm.at[didx_v.at[pl.ds(off * 2, DROWS)]],
                                  drows_v, sem_d).wait()
            for i in range(CHUNK):
                row = c * CHUNK + i + 1
                for h in range(NCOL):
                    s = pl.ds(h * 16, 16)
                    acc = arows_v[i * F, s]
                    for f in range(1, F):
                        acc = acc + arows_v[i * F + f, s]
                    acc = acc + drows_v[i * 2, s] + drows_v[i * 2 + 1, s]
                    out_v[row, s] = acc
            return carry2

        lax.fori_loop(0, CHUNKS_PER_BATCH, chunk_body, carry)
        b = wid * BATCHES_PER_W + lb
        pltpu.sync_copy(out_v, out_hbm.at[pl.ds(b * (N + 1), N + 1), :])
        return carry

    lax.fori_loop(0, BATCHES_PER_W, batch_body, 0)


@jax.jit
def _run(atom_table, degc, graph_token, aidx, didx):
    mesh = plsc.VectorSubcoreMesh(core_axis_name="c", subcore_axis_name="s")
    kfn = functools.partial(
        pl.kernel,
        mesh=mesh,
        out_type=jax.ShapeDtypeStruct((B * (N + 1), H), jnp.float32),
        scratch_types=[
            pltpu.VMEM((NODES_PER_W * F,), jnp.int32),
            pltpu.VMEM((NODES_PER_W * 2,), jnp.int32),
            pltpu.VMEM((AROWS, H), jnp.float32),
            pltpu.VMEM((DROWS, H), jnp.float32),
            pltpu.VMEM((N + 1, H), jnp.float32),
            pltpu.SemaphoreType.DMA,
            pltpu.SemaphoreType.DMA,
        ],
    )(_body)
    return kfn(atom_table, degc, graph_token, aidx, didx)


def kernel(x, in_degree, out_degree, atom_table, in_deg_table, out_deg_table,
           graph_token):
    degc = jnp.concatenate([in_deg_table, out_deg_table], axis=0)
    aidx = x.reshape(-1)
    didx = jnp.stack([in_degree, out_degree + NUM_IN_DEG], axis=-1).reshape(-1)
    out = _run(atom_table, degc, graph_token, aidx, didx)
    return out.reshape(B, N + 1, H)


# speedup vs baseline: 3.3400x; 3.3400x over previous
import functools

import jax
import jax.numpy as jnp
from jax import lax
from jax.experimental import pallas as pl
from jax.experimental.pallas import tpu as pltpu
from jax.experimental.pallas import tpu_sc as plsc

B, N, F, H = 256, 128, 9, 128
NUM_IN_DEG = 512

NW = 32
BATCHES_PER_W = B // NW
NODES_PER_W = BATCHES_PER_W * N
CHUNK = 8
CHUNKS_PER_BATCH = N // CHUNK
AROWS = CHUNK * F
DROWS = CHUNK * 2
NCOL = H // 16


def _body(atom_hbm, degc_hbm, gt_hbm, aidx_hbm, didx_hbm, out_hbm,
          aidx_v, didx_v, arows_v, drows_v, out_v, sem_a, sem_d):
    nc = 2
    wid = lax.axis_index("s") * nc + lax.axis_index("c")

    pltpu.sync_copy(aidx_hbm.at[pl.ds(wid * NODES_PER_W * F, NODES_PER_W * F)],
                    aidx_v)
    pltpu.sync_copy(didx_hbm.at[pl.ds(wid * NODES_PER_W * 2, NODES_PER_W * 2)],
                    didx_v)
    pltpu.sync_copy(gt_hbm, out_v.at[pl.ds(0, 1), :])

    def batch_body(lb, carry):
        def chunk_body(c, carry2):
            off = lb * N + c * CHUNK
            pltpu.async_copy(atom_hbm.at[aidx_v.at[pl.ds(off * F, AROWS)]],
                             arows_v, sem_a)
            pltpu.async_copy(degc_hbm.at[didx_v.at[pl.ds(off * 2, DROWS)]],
                             drows_v, sem_d)
            pltpu.make_async_copy(atom_hbm.at[aidx_v.at[pl.ds(off * F, AROWS)]],
                                  arows_v, sem_a).wait()
            pltpu.make_async_copy(degc_hbm.at[didx_v.at[pl.ds(off * 2, DROWS)]],
                                  drows_v, sem_d).wait()
            for i in range(CHUNK):
                row = c * CHUNK + i + 1
                for h in range(NCOL):
                    s = pl.ds(h * 16, 16)
                    acc = arows_v[i * F, s]
                    for f in range(1, F):
                        acc = acc + arows_v[i * F + f, s]
                    acc = acc + drows_v[i * 2, s] + drows_v[i * 2 + 1, s]
                    out_v[row, s] = acc
            return carry2

        lax.fori_loop(0, CHUNKS_PER_BATCH, chunk_body, carry)
        b = wid * BATCHES_PER_W + lb
        pltpu.sync_copy(out_v, out_hbm.at[b])
        return carry

    lax.fori_loop(0, BATCHES_PER_W, batch_body, 0)


@jax.jit
def _run(atom_table, degc, graph_token, aidx, didx):
    mesh = plsc.VectorSubcoreMesh(core_axis_name="c", subcore_axis_name="s")
    kfn = functools.partial(
        pl.kernel,
        mesh=mesh,
        out_type=jax.ShapeDtypeStruct((B, N + 1, H), jnp.float32),
        scratch_types=[
            pltpu.VMEM((NODES_PER_W * F,), jnp.int32),
            pltpu.VMEM((NODES_PER_W * 2,), jnp.int32),
            pltpu.VMEM((AROWS, H), jnp.float32),
            pltpu.VMEM((DROWS, H), jnp.float32),
            pltpu.VMEM((N + 1, H), jnp.float32),
            pltpu.SemaphoreType.DMA,
            pltpu.SemaphoreType.DMA,
        ],
    )(_body)
    return kfn(atom_table, degc, graph_token, aidx, didx)


def kernel(x, in_degree, out_degree, atom_table, in_deg_table, out_deg_table,
           graph_token):
    degc = jnp.concatenate([in_deg_table, out_deg_table], axis=0)
    aidx = x.reshape(-1)
    didx = jnp.stack([in_degree, out_degree + NUM_IN_DEG], axis=-1).reshape(-1)
    return _run(atom_table, degc, graph_token, aidx, didx)


# speedup vs baseline: 4.3938x; 1.3155x over previous
import functools

import jax
import jax.numpy as jnp
from jax import lax
from jax.experimental import pallas as pl
from jax.experimental.pallas import tpu as pltpu
from jax.experimental.pallas import tpu_sc as plsc

B, N, F, H = 256, 128, 9, 128
NUM_IN_DEG = 512

NW = 32
BATCHES_PER_W = B // NW
NODES_PER_W = BATCHES_PER_W * N
CHUNK = 8
CHUNKS_PER_BATCH = N // CHUNK
NCHUNKS = BATCHES_PER_W * CHUNKS_PER_BATCH
AROWS = CHUNK * F
DROWS = CHUNK * 2
NCOL = H // 16


def _body(atom_hbm, degc_hbm, gt_hbm, aidx_hbm, didx_hbm, out_hbm,
          aidx_v, didx_v, a0_v, a1_v, d0_v, d1_v, o0_v, o1_v, tok_v,
          sa0, sa1, sd0, sd1, so0, so1):
    nc = 2
    wid = lax.axis_index("s") * nc + lax.axis_index("c")

    pltpu.sync_copy(aidx_hbm.at[pl.ds(wid * NODES_PER_W * F, NODES_PER_W * F)],
                    aidx_v)
    pltpu.sync_copy(didx_hbm.at[pl.ds(wid * NODES_PER_W * 2, NODES_PER_W * 2)],
                    didx_v)
    pltpu.sync_copy(gt_hbm, tok_v)
    for h in range(NCOL):
        s = pl.ds(h * 16, 16)
        t = tok_v[0, s]
        o0_v[0, s] = t
        o1_v[0, s] = t

    def gather_desc(k, abuf, dbuf, sa, sd):
        da = pltpu.make_async_copy(
            atom_hbm.at[aidx_v.at[pl.ds(k * AROWS, AROWS)]], abuf, sa)
        dd = pltpu.make_async_copy(
            degc_hbm.at[didx_v.at[pl.ds(k * DROWS, DROWS)]], dbuf, sd)
        return da, dd

    def issue_gather(k, abuf, dbuf, sa, sd):
        da, dd = gather_desc(k, abuf, dbuf, sa, sd)
        da.start()
        dd.start()

    def wait_gather(k, abuf, dbuf, sa, sd):
        da, dd = gather_desc(k, abuf, dbuf, sa, sd)
        da.wait()
        dd.wait()

    def compute(c, abuf, dbuf, obuf):
        for i in range(CHUNK):
            for h in range(NCOL):
                s = pl.ds(h * 16, 16)
                t0 = abuf[i * F + 0, s] + abuf[i * F + 1, s]
                t1 = abuf[i * F + 2, s] + abuf[i * F + 3, s]
                t2 = abuf[i * F + 4, s] + abuf[i * F + 5, s]
                t3 = abuf[i * F + 6, s] + abuf[i * F + 7, s]
                t4 = abuf[i * F + 8, s] + dbuf[i * 2, s]
                t5 = dbuf[i * 2 + 1, s]
                obuf[c * CHUNK + i + 1, s] = ((t0 + t1) + (t2 + t3)) + (t4 + t5)

    def flush_desc(lb, obuf, so):
        return pltpu.make_async_copy(obuf,
                                     out_hbm.at[wid * BATCHES_PER_W + lb], so)

    def process_batch(lb, obuf, so):
        def pair_body(pp, carry):
            ka = lb * CHUNKS_PER_BATCH + 2 * pp
            kb = ka + 1
            kc = jnp.minimum(ka + 2, NCHUNKS - 1)
            issue_gather(kb, a1_v, d1_v, sa1, sd1)
            wait_gather(ka, a0_v, d0_v, sa0, sd0)
            compute(2 * pp, a0_v, d0_v, obuf)
            issue_gather(kc, a0_v, d0_v, sa0, sd0)
            wait_gather(kb, a1_v, d1_v, sa1, sd1)
            compute(2 * pp + 1, a1_v, d1_v, obuf)
            return carry

        lax.fori_loop(0, CHUNKS_PER_BATCH // 2, pair_body, 0)
        flush_desc(lb, obuf, so).start()

    issue_gather(0, a0_v, d0_v, sa0, sd0)

    def batch_pair_body(q, carry):
        @pl.when(q > 0)
        def _():
            flush_desc(2 * q, o0_v, so0).wait()

        process_batch(2 * q, o0_v, so0)

        @pl.when(q > 0)
        def _():
            flush_desc(2 * q + 1, o1_v, so1).wait()

        process_batch(2 * q + 1, o1_v, so1)
        return carry

    lax.fori_loop(0, BATCHES_PER_W // 2, batch_pair_body, 0)

    wait_gather(NCHUNKS - 1, a0_v, d0_v, sa0, sd0)
    flush_desc(0, o0_v, so0).wait()
    flush_desc(0, o1_v, so1).wait()


@jax.jit
def _run(atom_table, degc, graph_token, aidx, didx):
    mesh = plsc.VectorSubcoreMesh(core_axis_name="c", subcore_axis_name="s")
    kfn = functools.partial(
        pl.kernel,
        mesh=mesh,
        out_type=jax.ShapeDtypeStruct((B, N + 1, H), jnp.float32),
        scratch_types=[
            pltpu.VMEM((NODES_PER_W * F,), jnp.int32),
            pltpu.VMEM((NODES_PER_W * 2,), jnp.int32),
            pltpu.VMEM((AROWS, H), jnp.float32),
            pltpu.VMEM((AROWS, H), jnp.float32),
            pltpu.VMEM((DROWS, H), jnp.float32),
            pltpu.VMEM((DROWS, H), jnp.float32),
            pltpu.VMEM((N + 1, H), jnp.float32),
            pltpu.VMEM((N + 1, H), jnp.float32),
            pltpu.VMEM((1, H), jnp.float32),
            pltpu.SemaphoreType.DMA,
            pltpu.SemaphoreType.DMA,
            pltpu.SemaphoreType.DMA,
            pltpu.SemaphoreType.DMA,
            pltpu.SemaphoreType.DMA,
            pltpu.SemaphoreType.DMA,
        ],
    )(_body)
    return kfn(atom_table, degc, graph_token, aidx, didx)


def kernel(x, in_degree, out_degree, atom_table, in_deg_table, out_deg_table,
           graph_token):
    degc = jnp.concatenate([in_deg_table, out_deg_table], axis=0)
    aidx = x.reshape(-1)
    didx = jnp.stack([in_degree, out_degree + NUM_IN_DEG], axis=-1).reshape(-1)
    return _run(atom_table, degc, graph_token, aidx, didx)


# speedup vs baseline: 8.4323x; 1.9191x over previous
import functools

import jax
import jax.numpy as jnp
from jax import lax
from jax.experimental import pallas as pl
from jax.experimental.pallas import tpu as pltpu
from jax.experimental.pallas import tpu_sc as plsc

B, N, F, H = 256, 128, 9, 128
NUM_IN_DEG = 512

NW = 32
BATCHES_PER_W = B // NW
NODES_PER_W = BATCHES_PER_W * N
CHUNK = 8
CHUNKS_PER_BATCH = N // CHUNK
NCHUNKS = BATCHES_PER_W * CHUNKS_PER_BATCH
AROWS = CHUNK * F
DROWS = CHUNK * 2
NCOL = H // 16
NBUF = 4


def _body(atom_hbm, degc_hbm, gt_hbm, aidx_hbm, didx_hbm, out_hbm,
          aidx_v, didx_v, a_bufs, d_bufs, o0_v, o1_v, tok_v,
          sa, sd, so0, so1):
    nc = 2
    wid = lax.axis_index("s") * nc + lax.axis_index("c")

    pltpu.sync_copy(aidx_hbm.at[pl.ds(wid * NODES_PER_W * F, NODES_PER_W * F)],
                    aidx_v)
    pltpu.sync_copy(didx_hbm.at[pl.ds(wid * NODES_PER_W * 2, NODES_PER_W * 2)],
                    didx_v)
    pltpu.sync_copy(gt_hbm, tok_v)
    for h in range(NCOL):
        s = pl.ds(h * 16, 16)
        t = tok_v[0, s]
        o0_v[0, s] = t
        o1_v[0, s] = t

    def gather_desc(k, u):
        da = pltpu.make_async_copy(
            atom_hbm.at[aidx_v.at[pl.ds(k * AROWS, AROWS)]], a_bufs[u], sa[u])
        dd = pltpu.make_async_copy(
            degc_hbm.at[didx_v.at[pl.ds(k * DROWS, DROWS)]], d_bufs[u], sd[u])
        return da, dd

    def issue_gather(k, u):
        da, dd = gather_desc(k, u)
        da.start()
        dd.start()

    def wait_gather(k, u):
        da, dd = gather_desc(k, u)
        da.wait()
        dd.wait()

    def compute(c, u, obuf):
        abuf = a_bufs[u]
        dbuf = d_bufs[u]

        def node_body(i, carry):
            for h in range(NCOL):
                s = pl.ds(h * 16, 16)
                t0 = abuf[i * F + 0, s] + abuf[i * F + 1, s]
                t1 = abuf[i * F + 2, s] + abuf[i * F + 3, s]
                t2 = abuf[i * F + 4, s] + abuf[i * F + 5, s]
                t3 = abuf[i * F + 6, s] + abuf[i * F + 7, s]
                t4 = abuf[i * F + 8, s] + dbuf[i * 2, s]
                t5 = dbuf[i * 2 + 1, s]
                obuf[c * CHUNK + i + 1, s] = ((t0 + t1) + (t2 + t3)) + (t4 + t5)
            return carry

        lax.fori_loop(0, CHUNK, node_body, 0)

    def flush_desc(lb, obuf, so):
        return pltpu.make_async_copy(obuf,
                                     out_hbm.at[wid * BATCHES_PER_W + lb], so)

    def process_batch(lb, obuf, so):
        def quad_body(j, carry):
            kq = lb * CHUNKS_PER_BATCH + NBUF * j
            for u in range(NBUF):
                k = kq + u
                kn = jnp.minimum(k + (NBUF - 1), NCHUNKS - 1)
                issue_gather(kn, (u + NBUF - 1) % NBUF)
                wait_gather(k, u)
                compute(NBUF * j + u, u, obuf)
            return carry

        lax.fori_loop(0, CHUNKS_PER_BATCH // NBUF, quad_body, 0)
        flush_desc(lb, obuf, so).start()

    for u in range(NBUF - 1):
        issue_gather(u, u)

    def batch_pair_body(q, carry):
        @pl.when(q > 0)
        def _():
            flush_desc(2 * q, o0_v, so0).wait()

        process_batch(2 * q, o0_v, so0)

        @pl.when(q > 0)
        def _():
            flush_desc(2 * q + 1, o1_v, so1).wait()

        process_batch(2 * q + 1, o1_v, so1)
        return carry

    lax.fori_loop(0, BATCHES_PER_W // 2, batch_pair_body, 0)

    for u in range(NBUF - 1):
        wait_gather(NCHUNKS - 1, u)
    flush_desc(0, o0_v, so0).wait()
    flush_desc(0, o1_v, so1).wait()


@jax.jit
def _run(atom_table, degc, graph_token, aidx, didx):
    mesh = plsc.VectorSubcoreMesh(core_axis_name="c", subcore_axis_name="s")

    def body(atom_hbm, degc_hbm, gt_hbm, aidx_hbm, didx_hbm, out_hbm,
             aidx_v, didx_v, a0, a1, a2, a3, d0, d1, d2, d3, o0_v, o1_v,
             tok_v, sa0, sa1, sa2, sa3, sd0, sd1, sd2, sd3, so0, so1):
        _body(atom_hbm, degc_hbm, gt_hbm, aidx_hbm, didx_hbm, out_hbm,
              aidx_v, didx_v, (a0, a1, a2, a3), (d0, d1, d2, d3),
              o0_v, o1_v, tok_v, (sa0, sa1, sa2, sa3),
              (sd0, sd1, sd2, sd3), so0, so1)

    kfn = functools.partial(
        pl.kernel,
        mesh=mesh,
        out_type=jax.ShapeDtypeStruct((B, N + 1, H), jnp.float32),
        scratch_types=[
            pltpu.VMEM((NODES_PER_W * F,), jnp.int32),
            pltpu.VMEM((NODES_PER_W * 2,), jnp.int32),
        ] + [pltpu.VMEM((AROWS, H), jnp.float32)] * NBUF
        + [pltpu.VMEM((DROWS, H), jnp.float32)] * NBUF
        + [
            pltpu.VMEM((N + 1, H), jnp.float32),
            pltpu.VMEM((N + 1, H), jnp.float32),
            pltpu.VMEM((1, H), jnp.float32),
        ] + [pltpu.SemaphoreType.DMA] * (2 * NBUF + 2),
    )(body)
    return kfn(atom_table, degc, graph_token, aidx, didx)


def kernel(x, in_degree, out_degree, atom_table, in_deg_table, out_deg_table,
           graph_token):
    degc = jnp.concatenate([in_deg_table, out_deg_table], axis=0)
    aidx = x.reshape(-1)
    didx = jnp.stack([in_degree, out_degree + NUM_IN_DEG], axis=-1).reshape(-1)
    return _run(atom_table, degc, graph_token, aidx, didx)


# speedup vs baseline: 8.6341x; 1.0239x over previous
import functools

import jax
import jax.numpy as jnp
from jax import lax
from jax.experimental import pallas as pl
from jax.experimental.pallas import tpu as pltpu
from jax.experimental.pallas import tpu_sc as plsc

B, N, F, H = 256, 128, 9, 128
NUM_IN_DEG = 512

NW = 32
NSUB = 16
BATCHES_PER_W = B // NW
NODES_PER_W = BATCHES_PER_W * N
CHUNK = 8
CHUNKS_PER_BATCH = N // CHUNK
NCHUNKS = BATCHES_PER_W * CHUNKS_PER_BATCH
AROWS = CHUNK * F
DROWS = CHUNK * 2
NBUF = 4
SROW = 136
NSLOT = NSUB * 2


def _body(atom_hbm, degc_hbm, gt_hbm, aidx_hbm, didx_hbm, sc_hbm,
          out_hbm, aidx_v, didx_v, sc_v, a_bufs, ztok_v,
          stage_sh, sa, sd, ssa, si, so):
    nc = 2
    s_id = lax.axis_index("s")
    wid = s_id * nc + lax.axis_index("c")

    pltpu.sync_copy(aidx_hbm.at[pl.ds(wid * NODES_PER_W * F, NODES_PER_W * F)],
                    aidx_v)
    pltpu.sync_copy(didx_hbm.at[pl.ds(wid * NODES_PER_W * 2, NODES_PER_W * 2)],
                    didx_v)
    pltpu.sync_copy(sc_hbm.at[wid], sc_v)
    pltpu.sync_copy(gt_hbm, ztok_v.at[pl.ds(0, 1)])
    zv = jnp.zeros((16,), jnp.float32)

    def zrow(r, carry):
        for h in range(H // 16):
            ztok_v[r, pl.ds(h * 16, 16)] = zv
        return carry

    lax.fori_loop(1, N + 1, zrow, 0)

    def gather_desc(k, u):
        da = pltpu.make_async_copy(
            atom_hbm.at[aidx_v.at[pl.ds(k * AROWS, AROWS)]],
            a_bufs[u].at[pl.ds(0, AROWS)], sa[u])
        dd = pltpu.make_async_copy(
            degc_hbm.at[didx_v.at[pl.ds(k * DROWS, DROWS)]],
            a_bufs[u].at[pl.ds(AROWS, DROWS)], sd[u])
        return da, dd

    def issue_gather(k, u):
        da, dd = gather_desc(k, u)
        da.start()
        dd.start()

    def wait_gather(k, u):
        da, dd = gather_desc(k, u)
        da.wait()
        dd.wait()

    def issue_scatter(par, cloc, u):
        pltpu.async_copy(a_bufs[u], stage_sh.at[sc_v.at[par, cloc]], ssa[u],
                         add=True)

    def wait_scatter(u):
        pltpu.make_async_copy(a_bufs[u], stage_sh.at[sc_v.at[0, 0]],
                              ssa[u]).wait()

    def base_row(par):
        return (s_id * 2 + par) * SROW

    def init_desc(par):
        return pltpu.make_async_copy(
            ztok_v, stage_sh.at[pl.ds(base_row(par), N + 1)], si[par])

    def flush_desc(lb, par):
        return pltpu.make_async_copy(
            stage_sh.at[pl.ds(base_row(par), N + 1)],
            out_hbm.at[wid * BATCHES_PER_W + lb], so[par])

    def process_batch(q, par):
        lb = 2 * q + par

        @pl.when(q > 0)
        def _():
            flush_desc(lb, par).wait()

        init_desc(par).start()

        def quad_body(j, carry):
            kq = lb * CHUNKS_PER_BATCH + NBUF * j
            for u in range(NBUF):
                k = kq + u
                kn = jnp.minimum(k + (NBUF - 1), NCHUNKS - 1)
                un = (u + NBUF - 1) % NBUF
                if u == 0:
                    @pl.when(j > 0)
                    def _():
                        wait_scatter(un)
                else:
                    wait_scatter(un)
                issue_gather(kn, un)
                wait_gather(k, u)
                if u == 0:
                    @pl.when(j == 0)
                    def _():
                        init_desc(par).wait()
                issue_scatter(par, NBUF * j + u, u)
            return carry

        lax.fori_loop(0, CHUNKS_PER_BATCH // NBUF, quad_body, 0)
        wait_scatter(NBUF - 1)
        flush_desc(lb, par).start()

    for u in range(NBUF - 1):
        issue_gather(u, u)

    def batch_pair_body(q, carry):
        process_batch(q, 0)
        process_batch(q, 1)
        return carry

    lax.fori_loop(0, BATCHES_PER_W // 2, batch_pair_body, 0)

    for u in range(NBUF - 1):
        wait_gather(NCHUNKS - 1, u)
    flush_desc(0, 0).wait()
    flush_desc(0, 1).wait()


@jax.jit
def _run(atom_table, degc, graph_token, aidx, didx, sc):
    mesh = plsc.VectorSubcoreMesh(core_axis_name="c", subcore_axis_name="s")

    def body(atom_hbm, degc_hbm, gt_hbm, aidx_hbm, didx_hbm, sc_hbm,
             out_hbm, aidx_v, didx_v, sc_v, a0, a1, a2, a3, ztok_v, stage_sh,
             sa0, sa1, sa2, sa3, sd0, sd1, sd2, sd3,
             ssa0, ssa1, ssa2, ssa3, si0, si1, so0, so1):
        _body(atom_hbm, degc_hbm, gt_hbm, aidx_hbm, didx_hbm, sc_hbm,
              out_hbm, aidx_v, didx_v, sc_v, (a0, a1, a2, a3),
              ztok_v, stage_sh,
              (sa0, sa1, sa2, sa3), (sd0, sd1, sd2, sd3),
              (ssa0, ssa1, ssa2, ssa3), (si0, si1), (so0, so1))

    kfn = functools.partial(
        pl.kernel,
        mesh=mesh,
        out_type=jax.ShapeDtypeStruct((B, N + 1, H), jnp.float32),
        scratch_types=[
            pltpu.VMEM((NODES_PER_W * F,), jnp.int32),
            pltpu.VMEM((NODES_PER_W * 2,), jnp.int32),
            pltpu.VMEM((2, CHUNKS_PER_BATCH, AROWS + DROWS), jnp.int32),
        ] + [pltpu.VMEM((AROWS + DROWS, H), jnp.float32)] * NBUF
        + [
            pltpu.VMEM((N + 1, H), jnp.float32),
            pltpu.VMEM_SHARED((NSLOT * SROW, H), jnp.float32),
        ] + [pltpu.SemaphoreType.DMA] * (3 * NBUF + 4),
    )(body)
    return kfn(atom_table, degc, graph_token, aidx, didx, sc)


def kernel(x, in_degree, out_degree, atom_table, in_deg_table, out_deg_table,
           graph_token):
    degc = jnp.concatenate([in_deg_table, out_deg_table], axis=0)
    aidx = x.reshape(-1)
    didx = jnp.stack([in_degree, out_degree + NUM_IN_DEG], axis=-1).reshape(-1)
    sub = jnp.arange(NW, dtype=jnp.int32) // 2
    node = jnp.arange(N, dtype=jnp.int32).reshape(CHUNKS_PER_BATCH, CHUNK)
    par = jnp.arange(2, dtype=jnp.int32)
    base = (sub[:, None] * 2 + par[None, :]) * SROW
    arow = 1 + jnp.repeat(node, F, axis=1)
    drow = 1 + jnp.repeat(node, 2, axis=1)
    row = jnp.concatenate([arow, drow], axis=1)
    sc = base[:, :, None, None] + row[None, None]
    return _run(atom_table, degc, graph_token, aidx, didx, sc)
